# baseline (speedup 1.0000x reference)
import functools
import math

import jax
import jax.numpy as jnp
from jax import lax
from jax.experimental import pallas as pl
from jax.experimental.pallas import tpu as pltpu
from jax.experimental.pallas import tpu_sc as plsc

D_MODEL = 64
SCALE = math.sqrt(D_MODEL)
CHUNK = 128
LANES = 16


def _emb_body(n_per_w, n_chunks, num_cores, x_hbm, lut_hbm, out_hbm,
              idx_v, rows_v, sem):
    wid = lax.axis_index("s") * num_cores + lax.axis_index("c")
    base = wid * n_per_w

    def chunk_body(c, _):
        off = base + c * CHUNK
        pltpu.sync_copy(x_hbm.at[pl.ds(off, CHUNK)], idx_v)
        pltpu.async_copy(lut_hbm.at[idx_v], rows_v, sem).wait()

        def mul_body(i, _):
            for r in range(4):
                row = i * 4 + r
                for j in range(D_MODEL // LANES):
                    sl = pl.ds(j * LANES, LANES)
                    rows_v[row, sl] = rows_v[row, sl] * SCALE
            return 0

        lax.fori_loop(0, CHUNK // 4, mul_body, 0)
        pltpu.sync_copy(rows_v, out_hbm.at[pl.ds(off, CHUNK)])
        return 0

    lax.fori_loop(0, n_chunks, chunk_body, 0)


def kernel(x, lut):
    b, t = x.shape
    n = b * t
    flat_x = x.reshape(n).astype(jnp.int32)

    info = plsc.get_sparse_core_info()
    num_workers = info.num_cores * info.num_subcores
    n_per_w = n // num_workers
    assert n_per_w * num_workers == n
    n_chunks = n_per_w // CHUNK
    assert n_chunks * CHUNK == n_per_w

    mesh = plsc.VectorSubcoreMesh(core_axis_name="c", subcore_axis_name="s")
    body = functools.partial(_emb_body, n_per_w, n_chunks, info.num_cores)

    out = pl.kernel(
        body,
        mesh=mesh,
        compiler_params=pltpu.CompilerParams(use_tc_tiling_on_sc=False),
        out_type=jax.ShapeDtypeStruct((n, D_MODEL), jnp.float32),
        scratch_types=[
            pltpu.VMEM((CHUNK,), jnp.int32),
            pltpu.VMEM((CHUNK, D_MODEL), jnp.float32),
            pltpu.SemaphoreType.DMA,
        ],
    )(flat_x, lut)
    return out.reshape(b, t, D_MODEL)


# speedup vs baseline: 1.2294x; 1.2294x over previous
import functools
import math

import jax
import jax.numpy as jnp
from jax import lax
from jax.experimental import pallas as pl
from jax.experimental.pallas import tpu as pltpu
from jax.experimental.pallas import tpu_sc as plsc

D_MODEL = 64
SCALE = math.sqrt(D_MODEL)
LANES = 16
G = 512
GSUB = 128
IW = G // GSUB
NBUF = 2


def _emb_body(n_per_w, num_cores, x_hbm, lut_hbm, out_hbm,
              idx0, idx1, rows0, rows1, si0, si1, sg0, sg1, so0, so1):
    idx = (idx0, idx1)
    rows = (rows0, rows1)
    si = (si0, si1)
    sg = (sg0, sg1)
    so = (so0, so1)

    wid = lax.axis_index("s") * num_cores + lax.axis_index("c")
    base = wid * n_per_w
    xrow = wid * (n_per_w // GSUB)
    ng = n_per_w // G
    n_outer = ng // NBUF

    for b in range(NBUF):
        pltpu.async_copy(x_hbm.at[pl.ds(xrow + b * IW, IW)], idx[b], si[b])

    def outer(gg, _):
        for b in range(NBUF):
            @pl.when(gg > 0)
            def _wait_out():
                pltpu.make_async_copy(rows[b], out_hbm.at[pl.ds(base, G)],
                                      so[b]).wait()
            pltpu.make_async_copy(x_hbm.at[pl.ds(xrow, IW)], idx[b],
                                  si[b]).wait()
            for j in range(IW):
                pltpu.async_copy(lut_hbm.at[idx[b].at[j]],
                                 rows[b].at[pl.ds(j * GSUB, GSUB)], sg[b])
        for b in range(NBUF):
            g = gg * NBUF + b
            pltpu.make_async_copy(lut_hbm.at[pl.ds(0, G)], rows[b],
                                  sg[b]).wait()

            @pl.when(gg < n_outer - 1)
            def _refill_idx():
                pltpu.async_copy(
                    x_hbm.at[pl.ds(xrow + (g + NBUF) * IW, IW)],
                    idx[b], si[b])

            def mul(i, _):
                for r in range(8):
                    row = i * 8 + r
                    for q in range(D_MODEL // LANES):
                        sl = pl.ds(q * LANES, LANES)
                        rows[b][row, sl] = rows[b][row, sl] * SCALE
                return 0

            lax.fori_loop(0, G // 8, mul, 0)
            pltpu.async_copy(rows[b], out_hbm.at[pl.ds(base + g * G, G)],
                             so[b])
        return 0

    lax.fori_loop(0, n_outer, outer, 0)
    for b in range(NBUF):
        pltpu.make_async_copy(rows[b], out_hbm.at[pl.ds(base, G)],
                              so[b]).wait()


def kernel(x, lut):
    b, t = x.shape
    n = b * t
    x2d = x.reshape(n // GSUB, GSUB).astype(jnp.int32)

    info = plsc.get_sparse_core_info()
    num_workers = info.num_cores * info.num_subcores
    n_per_w = n // num_workers
    assert n_per_w * num_workers == n
    assert n_per_w % (G * NBUF) == 0

    mesh = plsc.VectorSubcoreMesh(core_axis_name="c", subcore_axis_name="s")
    body = functools.partial(_emb_body, n_per_w, info.num_cores)

    out = pl.kernel(
        body,
        mesh=mesh,
        compiler_params=pltpu.CompilerParams(use_tc_tiling_on_sc=False),
        out_type=jax.ShapeDtypeStruct((n, D_MODEL), jnp.float32),
        scratch_types=[
            pltpu.VMEM((IW, GSUB), jnp.int32),
            pltpu.VMEM((IW, GSUB), jnp.int32),
            pltpu.VMEM((G, D_MODEL), jnp.float32),
            pltpu.VMEM((G, D_MODEL), jnp.float32),
            pltpu.SemaphoreType.DMA,
            pltpu.SemaphoreType.DMA,
            pltpu.SemaphoreType.DMA,
            pltpu.SemaphoreType.DMA,
            pltpu.SemaphoreType.DMA,
            pltpu.SemaphoreType.DMA,
        ],
    )(x2d, lut)
    return out.reshape(b, t, D_MODEL)
